# baseline (speedup 1.0000x reference)
import functools

import jax
import jax.numpy as jnp
from jax import lax
from jax.experimental import pallas as pl
from jax.experimental.pallas import tpu as pltpu
from jax.experimental.pallas import tpu_sc as plsc

N = 10000
D = 128
G = 64
C = 10

NC = 2
NS = 16
NW = NC * NS

NPAD = 10240
CHUNK = 128
HALF = CHUNK // 2
CPW = 81
EPW = CHUNK * CPW
EPAD = NW * EPW
RPS = NPAD // NS
NBUF = 3
GROUPS = CPW // NBUF


@functools.cache
def _build_sc_agg():
    mesh = plsc.VectorSubcoreMesh(
        core_axis_name="c", subcore_axis_name="s",
        num_cores=NC, num_subcores=NS)
    return pl.kernel(
        _sc_agg_body,
        out_type=jax.ShapeDtypeStruct((NC, NPAD, D), jnp.float32),
        mesh=mesh,
        compiler_params=pltpu.CompilerParams(use_tc_tiling_on_sc=False),
        scratch_types=[
            pltpu.VMEM((CPW, D // 2), jnp.int32),
            [pltpu.VMEM((CHUNK,), jnp.int32) for _ in range(NBUF)],
            [[pltpu.VMEM((HALF,), jnp.int32) for _ in range(2)]
             for _ in range(NBUF)],
            [pltpu.VMEM((CHUNK,), jnp.float32) for _ in range(NBUF)],
            [pltpu.VMEM((CHUNK, D // 2), jnp.int32) for _ in range(NBUF)],
            [pltpu.VMEM((HALF, D), jnp.float32) for _ in range(2)],
            pltpu.VMEM_SHARED((NPAD, D), jnp.float32),
            [pltpu.SemaphoreType.DMA for _ in range(NBUF)],
            [pltpu.SemaphoreType.DMA for _ in range(NBUF)],
            [pltpu.SemaphoreType.DMA for _ in range(2)],
        ],
    )


def _sc_agg(h_pack, src16, dst, w):
    return _build_sc_agg()(h_pack, src16, dst, w)


def _pack_rows(h):
    hb = h.astype(jnp.bfloat16).reshape(NPAD, D // 2, 2)
    return jax.lax.bitcast_convert_type(hb, jnp.int32)


def _sc_agg_body(h_hbm, src_hbm, dst_hbm, w_hbm, out_hbm,
                 src_v, idx_bufs, dst_bufs, w_bufs, rows_bufs, stages,
                 agg_sh, gsems, asems, ssems):
    cid = lax.axis_index("c")
    sid = lax.axis_index("s")
    wslot = cid * NS + sid

    pltpu.sync_copy(src_hbm.at[wslot], src_v)

    def zbody(r, _):
        for c in range(D // 16):
            stages[0][r, pl.ds(c * 16, 16)] = jnp.zeros((16,), jnp.float32)
        return 0
    lax.fori_loop(0, HALF, zbody, 0)
    base = sid * RPS
    for t in range(RPS // HALF):
        pltpu.sync_copy(stages[0], agg_sh.at[pl.ds(base + t * HALF, HALF)])
    plsc.subcore_barrier()

    mask_lo = jnp.full((16,), 65535, jnp.int32)

    def expand_src(j, b):
        for g in range(D // 32):
            v = src_v[j, pl.ds(g * 16, 16)]
            idx_bufs[b][pl.ds(g * 16, 16)] = v & mask_lo
            idx_bufs[b][pl.ds(HALF + g * 16, 16)] = \
                lax.shift_right_logical(v, 16)

    mask_hi = jnp.full((16,), -65536, jnp.int32)

    def unpack_scale(w_b, rows_v, half):
        stage = stages[half]

        def gbody(g, _):
            w16 = w_b[pl.ds(half * HALF + g * 16, 16)]
            for le in range(16):
                wb = w16.at[jnp.full((16,), le, jnp.int32)].get(
                    mode="promise_in_bounds")
                r = g * 16 + le
                for c in range(D // 32):
                    v = rows_v[half * HALF + r, pl.ds(c * 16, 16)]
                    ev = lax.bitcast_convert_type(
                        lax.shift_left(v, 16), jnp.float32)
                    od = lax.bitcast_convert_type(v & mask_hi, jnp.float32)
                    stage[r, pl.ds(c * 16, 16)] = ev * wb
                    stage[r, pl.ds(D // 2 + c * 16, 16)] = od * wb
            return 0
        lax.fori_loop(0, HALF // 16, gbody, 0)

    def fire(j, b):
        expand_src(j, b)
        pltpu.async_copy(dst_hbm.at[wslot, j, 0], dst_bufs[b][0], asems[b])
        pltpu.async_copy(dst_hbm.at[wslot, j, 1], dst_bufs[b][1], asems[b])
        pltpu.async_copy(w_hbm.at[wslot, j], w_bufs[b], asems[b])
        pltpu.async_copy(h_hbm.at[idx_bufs[b]], rows_bufs[b], gsems[b])

    for b in range(NBUF):
        fire(b, b)

    def group_body(g, _):
        for b in range(NBUF):
            j = g * NBUF + b
            pltpu.make_async_copy(
                dst_hbm.at[wslot, 0, 0], dst_bufs[b][0], asems[b]).wait()
            pltpu.make_async_copy(
                dst_hbm.at[wslot, 0, 1], dst_bufs[b][1], asems[b]).wait()
            pltpu.make_async_copy(
                w_hbm.at[wslot, 0], w_bufs[b], asems[b]).wait()
            pltpu.make_async_copy(
                h_hbm.at[pl.ds(0, CHUNK)], rows_bufs[b], gsems[b]).wait()
            for half in range(2):
                @pl.when(j >= 1)
                def _():
                    pltpu.make_async_copy(
                        out_hbm.at[0, pl.ds(0, HALF)],
                        stages[half], ssems[half]).wait()
                unpack_scale(w_bufs[b], rows_bufs[b], half)
                pltpu.async_copy(stages[half],
                                 agg_sh.at[dst_bufs[b][half]],
                                 ssems[half], add=True)

            @pl.when(g < GROUPS - 1)
            def _():
                fire(j + NBUF, b)
        return 0
    lax.fori_loop(0, GROUPS, group_body, 0)

    for half in range(2):
        pltpu.make_async_copy(
            out_hbm.at[0, pl.ds(0, HALF)], stages[half], ssems[half]).wait()

    plsc.subcore_barrier()
    pltpu.sync_copy(agg_sh.at[pl.ds(base, RPS)],
                    out_hbm.at[cid, pl.ds(base, RPS)])


def _tc_layer_body(agg_ref, h_ref, wrel_ref, wroot_ref, brel_ref, out_ref):
    a = agg_ref[0] + agg_ref[1]
    acc = jnp.dot(a, wrel_ref[...], preferred_element_type=jnp.float32)
    acc = acc + jnp.dot(h_ref[...], wroot_ref[...],
                        preferred_element_type=jnp.float32)
    out_ref[...] = jnp.maximum(acc + brel_ref[...], 0.0)


_TC_RB = 2560


def _tc_layer(agg, h, wrel, wroot, brel):
    grid = NPAD // _TC_RB
    return pl.pallas_call(
        _tc_layer_body,
        grid=(grid,),
        in_specs=[
            pl.BlockSpec((NC, _TC_RB, D), lambda i: (0, i, 0)),
            pl.BlockSpec((_TC_RB, D), lambda i: (i, 0)),
            pl.BlockSpec((D, D), lambda i: (0, 0)),
            pl.BlockSpec((D, D), lambda i: (0, 0)),
            pl.BlockSpec((1, D), lambda i: (0, 0)),
        ],
        out_specs=pl.BlockSpec((_TC_RB, D), lambda i: (i, 0)),
        out_shape=jax.ShapeDtypeStruct((NPAD, D), jnp.float32),
    )(agg, h, wrel, wroot, brel)


_FB = 1280


def _tc_final_body(agg_ref, h_ref, batch_ref, wrel_ref, wroot_ref, brel_ref,
                   wlin_ref, blin_ref, out_ref, pool_acc, cnt_acc):
    i = pl.program_id(0)
    ni = pl.num_programs(0)
    a = agg_ref[0] + agg_ref[1]
    acc = jnp.dot(a, wrel_ref[...], preferred_element_type=jnp.float32)
    acc = acc + jnp.dot(h_ref[...], wroot_ref[...],
                        preferred_element_type=jnp.float32)
    h3 = jnp.maximum(acc + brel_ref[...], 0.0)

    b = batch_ref[...][:, 0]
    gids = lax.broadcasted_iota(jnp.int32, (G, _FB), 0)
    oh = (b[None, :] == gids).astype(jnp.float32)
    p = jnp.dot(oh, h3, preferred_element_type=jnp.float32)
    cnt = jnp.broadcast_to(jnp.sum(oh, axis=1, keepdims=True), (G, D))

    @pl.when(i == 0)
    def _():
        pool_acc[...] = p
        cnt_acc[...] = cnt

    @pl.when(i > 0)
    def _():
        pool_acc[...] = pool_acc[...] + p
        cnt_acc[...] = cnt_acc[...] + cnt

    @pl.when(i == ni - 1)
    def _():
        pooled = pool_acc[...] / jnp.maximum(cnt_acc[...], 1.0)
        out_ref[...] = jnp.dot(pooled, wlin_ref[...],
                               preferred_element_type=jnp.float32) + blin_ref[...]


def _tc_final(agg, h, batch2d, wrel, wroot, brel, wlin_pad, blin_pad):
    grid = NPAD // _FB
    return pl.pallas_call(
        _tc_final_body,
        grid=(grid,),
        in_specs=[
            pl.BlockSpec((NC, _FB, D), lambda i: (0, i, 0)),
            pl.BlockSpec((_FB, D), lambda i: (i, 0)),
            pl.BlockSpec((_FB, 1), lambda i: (i, 0)),
            pl.BlockSpec((D, D), lambda i: (0, 0)),
            pl.BlockSpec((D, D), lambda i: (0, 0)),
            pl.BlockSpec((1, D), lambda i: (0, 0)),
            pl.BlockSpec((D, D), lambda i: (0, 0)),
            pl.BlockSpec((1, D), lambda i: (0, 0)),
        ],
        out_specs=pl.BlockSpec((G, D), lambda i: (0, 0)),
        out_shape=jax.ShapeDtypeStruct((G, D), jnp.float32),
        scratch_shapes=[
            pltpu.VMEM((G, D), jnp.float32),
            pltpu.VMEM((G, D), jnp.float32),
        ],
    )(agg, h, batch2d, wrel, wroot, brel, wlin_pad, blin_pad)


def kernel(x, edge_index, edge_attr, batch,
           W_rel1, b_rel1, W_root1,
           W_rel2, b_rel2, W_root2,
           W_rel3, b_rel3, W_root3,
           W_lin, b_lin):
    E = edge_index.shape[1]
    h0 = jnp.pad(x, ((0, NPAD - N), (0, 0)))
    src = jnp.pad(edge_index[0], (0, EPAD - E)).reshape(NW, CPW, HALF, 2)
    dst = jnp.pad(edge_index[1], (0, EPAD - E),
                  constant_values=N).reshape(NW, CPW, HALF, 2)
    w = jnp.pad(edge_attr, (0, EPAD - E)).reshape(NW, CPW, HALF, 2)
    src16 = src[..., 0] | (src[..., 1] << 16)
    dst2 = jnp.stack([dst[..., 0], dst[..., 1]], axis=2)
    wp = jnp.concatenate([w[..., 0], w[..., 1]], axis=-1)
    batch2d = jnp.pad(batch, (0, NPAD - N), constant_values=-1).reshape(
        NPAD, 1)
    brel1 = b_rel1.reshape(1, D)
    brel2 = b_rel2.reshape(1, D)
    brel3 = b_rel3.reshape(1, D)
    wlin_pad = jnp.pad(W_lin, ((0, 0), (0, D - C)))
    blin_pad = jnp.pad(b_lin, (0, D - C)).reshape(1, D)
    perm = jnp.concatenate([jnp.arange(0, D, 2), jnp.arange(1, D, 2)])
    wrel1p = W_rel1[perm]
    wrel2p = W_rel2[perm]
    wrel3p = W_rel3[perm]

    agg = _sc_agg(_pack_rows(h0), src16, dst2, wp)
    h1 = _tc_layer(agg, h0, wrel1p, W_root1, brel1)
    agg = _sc_agg(_pack_rows(h1), src16, dst2, wp)
    h2 = _tc_layer(agg, h1, wrel2p, W_root2, brel2)
    agg = _sc_agg(_pack_rows(h2), src16, dst2, wp)
    out128 = _tc_final(agg, h2, batch2d, wrel3p, W_root3, brel3,
                       wlin_pad, blin_pad)
    return out128[:, :C]


# speedup vs baseline: 1.7546x; 1.7546x over previous
import functools

import jax
import jax.numpy as jnp
from jax import lax
from jax.experimental import pallas as pl
from jax.experimental.pallas import tpu as pltpu
from jax.experimental.pallas import tpu_sc as plsc

N = 10000
D = 128
G = 64
C = 10

NC = 2
NS = 16
NW = NC * NS

NPAD = 10240
CHUNK = 128
CPW = 80
EPW = CHUNK * CPW
EPAD = NW * EPW
RPS = NPAD // NS
NBUF = 2
GROUPS = CPW // NBUF

@functools.cache
def _build_sc_agg():
    mesh = plsc.VectorSubcoreMesh(
        core_axis_name="c", subcore_axis_name="s",
        num_cores=NC, num_subcores=NS)
    return pl.kernel(
        _sc_agg_body,
        out_type=jax.ShapeDtypeStruct((NC, NPAD, D), jnp.float32),
        mesh=mesh,
        compiler_params=pltpu.CompilerParams(use_tc_tiling_on_sc=False),
        scratch_types=[
            pltpu.VMEM((CPW, CHUNK), jnp.int32),
            [pltpu.VMEM((CHUNK,), jnp.int32) for _ in range(NBUF)],
            [pltpu.VMEM((CHUNK,), jnp.float32) for _ in range(NBUF)],
            [pltpu.VMEM((CHUNK, D // 2), jnp.int32) for _ in range(NBUF)],
            pltpu.VMEM((CHUNK, D), jnp.float32),
            pltpu.VMEM_SHARED((NPAD, D), jnp.float32),
            [pltpu.SemaphoreType.DMA for _ in range(NBUF)],
            [pltpu.SemaphoreType.DMA for _ in range(NBUF)],
        ],
    )


def _sc_agg(h_pack, src, dst, w):
    return _build_sc_agg()(h_pack, src, dst, w)


def _pack_rows(h):
    hb = h.astype(jnp.bfloat16).reshape(NPAD, D // 2, 2)
    return jax.lax.bitcast_convert_type(hb, jnp.int32)


def _sc_agg_body(h_hbm, src_hbm, dst_hbm, w_hbm, out_hbm,
                 src_v, dst_bufs, w_bufs, rows_bufs, stage_v,
                 agg_sh, gsems, asems):
    cid = lax.axis_index("c")
    sid = lax.axis_index("s")
    wslot = cid * NS + sid

    pltpu.sync_copy(src_hbm.at[wslot], src_v)

    def zbody(r, _):
        for c in range(D // 16):
            stage_v[r, pl.ds(c * 16, 16)] = jnp.zeros((16,), jnp.float32)
        return 0
    lax.fori_loop(0, CHUNK, zbody, 0)
    base = sid * RPS
    for t in range(RPS // CHUNK):
        pltpu.sync_copy(stage_v, agg_sh.at[pl.ds(base + t * CHUNK, CHUNK)])
    plsc.subcore_barrier()

    def unpack_scale(w_b, rows_v):
        mask_hi = jnp.full((16,), -65536, jnp.int32)

        def gbody(g, _):
            w16 = w_b[pl.ds(g * 16, 16)]
            for le in range(16):
                wb = w16.at[jnp.full((16,), le, jnp.int32)].get(
                    mode="promise_in_bounds")
                r = g * 16 + le
                for c in range(D // 32):
                    v = rows_v[r, pl.ds(c * 16, 16)]
                    ev = lax.bitcast_convert_type(
                        lax.shift_left(v, 16), jnp.float32)
                    od = lax.bitcast_convert_type(v & mask_hi, jnp.float32)
                    stage_v[r, pl.ds(c * 16, 16)] = ev * wb
                    stage_v[r, pl.ds(D // 2 + c * 16, 16)] = od * wb
            return 0
        lax.fori_loop(0, CHUNK // 16, gbody, 0)

    def fire(j, b):
        pltpu.async_copy(dst_hbm.at[wslot, j], dst_bufs[b], asems[b])
        pltpu.async_copy(w_hbm.at[wslot, j], w_bufs[b], asems[b])
        pltpu.async_copy(h_hbm.at[src_v.at[j]], rows_bufs[b], gsems[b])

    for b in range(NBUF):
        fire(b, b)

    def group_body(g, _):
        for b in range(NBUF):
            j = g * NBUF + b
            pltpu.make_async_copy(
                dst_hbm.at[wslot, 0], dst_bufs[b], asems[b]).wait()
            pltpu.make_async_copy(
                w_hbm.at[wslot, 0], w_bufs[b], asems[b]).wait()
            pltpu.make_async_copy(
                h_hbm.at[pl.ds(0, CHUNK)], rows_bufs[b], gsems[b]).wait()
            unpack_scale(w_bufs[b], rows_bufs[b])

            @pl.when(g < GROUPS - 1)
            def _():
                pltpu.async_copy(w_hbm.at[wslot, j + NBUF], w_bufs[b],
                                 asems[b])
                pltpu.async_copy(h_hbm.at[src_v.at[j + NBUF]], rows_bufs[b],
                                 gsems[b])
            pltpu.sync_copy(stage_v, agg_sh.at[dst_bufs[b]], add=True)

            @pl.when(g < GROUPS - 1)
            def _():
                pltpu.async_copy(dst_hbm.at[wslot, j + NBUF], dst_bufs[b],
                                 asems[b])
        return 0
    lax.fori_loop(0, GROUPS, group_body, 0)

    plsc.subcore_barrier()
    pltpu.sync_copy(agg_sh.at[pl.ds(base, RPS)],
                    out_hbm.at[cid, pl.ds(base, RPS)])


def _tc_layer_body(agg_ref, h_ref, wrel_ref, wroot_ref, brel_ref, out_ref):
    a = agg_ref[0] + agg_ref[1]
    acc = jnp.dot(a, wrel_ref[...], preferred_element_type=jnp.float32)
    acc = acc + jnp.dot(h_ref[...], wroot_ref[...],
                        preferred_element_type=jnp.float32)
    out_ref[...] = jnp.maximum(acc + brel_ref[...], 0.0)


_TC_RB = 2560


def _tc_layer(agg, h, wrel, wroot, brel):
    grid = NPAD // _TC_RB
    return pl.pallas_call(
        _tc_layer_body,
        grid=(grid,),
        in_specs=[
            pl.BlockSpec((NC, _TC_RB, D), lambda i: (0, i, 0)),
            pl.BlockSpec((_TC_RB, D), lambda i: (i, 0)),
            pl.BlockSpec((D, D), lambda i: (0, 0)),
            pl.BlockSpec((D, D), lambda i: (0, 0)),
            pl.BlockSpec((1, D), lambda i: (0, 0)),
        ],
        out_specs=pl.BlockSpec((_TC_RB, D), lambda i: (i, 0)),
        out_shape=jax.ShapeDtypeStruct((NPAD, D), jnp.float32),
    )(agg, h, wrel, wroot, brel)


_FB = 1280


def _tc_final_body(agg_ref, h_ref, batch_ref, wrel_ref, wroot_ref, brel_ref,
                   wlin_ref, blin_ref, out_ref, pool_acc, cnt_acc):
    i = pl.program_id(0)
    ni = pl.num_programs(0)
    a = agg_ref[0] + agg_ref[1]
    acc = jnp.dot(a, wrel_ref[...], preferred_element_type=jnp.float32)
    acc = acc + jnp.dot(h_ref[...], wroot_ref[...],
                        preferred_element_type=jnp.float32)
    h3 = jnp.maximum(acc + brel_ref[...], 0.0)

    b = batch_ref[...][:, 0]
    gids = lax.broadcasted_iota(jnp.int32, (G, _FB), 0)
    oh = (b[None, :] == gids).astype(jnp.float32)
    p = jnp.dot(oh, h3, preferred_element_type=jnp.float32)
    cnt = jnp.broadcast_to(jnp.sum(oh, axis=1, keepdims=True), (G, D))

    @pl.when(i == 0)
    def _():
        pool_acc[...] = p
        cnt_acc[...] = cnt

    @pl.when(i > 0)
    def _():
        pool_acc[...] = pool_acc[...] + p
        cnt_acc[...] = cnt_acc[...] + cnt

    @pl.when(i == ni - 1)
    def _():
        pooled = pool_acc[...] / jnp.maximum(cnt_acc[...], 1.0)
        out_ref[...] = jnp.dot(pooled, wlin_ref[...],
                               preferred_element_type=jnp.float32) + blin_ref[...]


def _tc_final(agg, h, batch2d, wrel, wroot, brel, wlin_pad, blin_pad):
    grid = NPAD // _FB
    return pl.pallas_call(
        _tc_final_body,
        grid=(grid,),
        in_specs=[
            pl.BlockSpec((NC, _FB, D), lambda i: (0, i, 0)),
            pl.BlockSpec((_FB, D), lambda i: (i, 0)),
            pl.BlockSpec((_FB, 1), lambda i: (i, 0)),
            pl.BlockSpec((D, D), lambda i: (0, 0)),
            pl.BlockSpec((D, D), lambda i: (0, 0)),
            pl.BlockSpec((1, D), lambda i: (0, 0)),
            pl.BlockSpec((D, D), lambda i: (0, 0)),
            pl.BlockSpec((1, D), lambda i: (0, 0)),
        ],
        out_specs=pl.BlockSpec((G, D), lambda i: (0, 0)),
        out_shape=jax.ShapeDtypeStruct((G, D), jnp.float32),
        scratch_shapes=[
            pltpu.VMEM((G, D), jnp.float32),
            pltpu.VMEM((G, D), jnp.float32),
        ],
    )(agg, h, batch2d, wrel, wroot, brel, wlin_pad, blin_pad)


def kernel(x, edge_index, edge_attr, batch,
           W_rel1, b_rel1, W_root1,
           W_rel2, b_rel2, W_root2,
           W_rel3, b_rel3, W_root3,
           W_lin, b_lin):
    h0 = jnp.pad(x, ((0, NPAD - N), (0, 0)))
    src = jnp.pad(edge_index[0], (0, EPAD - edge_index.shape[1])).reshape(
        NW, CPW, CHUNK)
    dst = jnp.pad(edge_index[1], (0, EPAD - edge_index.shape[1]),
                  constant_values=N).reshape(NW, CPW, CHUNK)
    w = jnp.pad(edge_attr, (0, EPAD - edge_attr.shape[0])).reshape(
        NW, CPW, CHUNK)
    batch2d = jnp.pad(batch, (0, NPAD - N), constant_values=-1).reshape(
        NPAD, 1)
    brel1 = b_rel1.reshape(1, D)
    brel2 = b_rel2.reshape(1, D)
    brel3 = b_rel3.reshape(1, D)
    wlin_pad = jnp.pad(W_lin, ((0, 0), (0, D - C)))
    blin_pad = jnp.pad(b_lin, (0, D - C)).reshape(1, D)
    perm = jnp.concatenate([jnp.arange(0, D, 2), jnp.arange(1, D, 2)])
    wrel1p = W_rel1[perm]
    wrel2p = W_rel2[perm]
    wrel3p = W_rel3[perm]

    agg = _sc_agg(_pack_rows(h0), src, dst, w)
    h1 = _tc_layer(agg, h0, wrel1p, W_root1, brel1)
    agg = _sc_agg(_pack_rows(h1), src, dst, w)
    h2 = _tc_layer(agg, h1, wrel2p, W_root2, brel2)
    agg = _sc_agg(_pack_rows(h2), src, dst, w)
    out128 = _tc_final(agg, h2, batch2d, wrel3p, W_root3, brel3,
                       wlin_pad, blin_pad)
    return out128[:, :C]
